# baseline (speedup 1.0000x reference)
import functools

import jax
import jax.numpy as jnp
from jax import lax
from jax.experimental import pallas as pl
from jax.experimental.pallas import tpu as pltpu
from jax.experimental.pallas import tpu_sc as plsc

NC = 2
NS = 16
NW = NC * NS

GW = 128
K = 8
CH = K * GW
NBUF = 2


@functools.partial(jax.jit, static_argnums=(2, 3))
def _embedding_gather(tokens_flat, table, b_per_w, n_chunks):
    B = tokens_flat.shape[0]
    D = table.shape[1]

    mesh = plsc.VectorSubcoreMesh(core_axis_name="c", subcore_axis_name="s")

    @functools.partial(
        pl.kernel,
        out_type=jax.ShapeDtypeStruct((B, D), jnp.float32),
        mesh=mesh,
        scratch_types=[
            pltpu.VMEM((NBUF, CH), jnp.int32),
            pltpu.VMEM((NBUF, CH, D), jnp.float32),
            pltpu.SemaphoreType.DMA((NBUF,)),
            pltpu.SemaphoreType.DMA((NBUF,)),
            pltpu.SemaphoreType.DMA((NBUF,)),
        ],
        compiler_params=pltpu.CompilerParams(
            use_tc_tiling_on_sc=False, needs_layout_passes=False
        ),
    )
    def k(tok_hbm, table_hbm, out_hbm, idx_v, rows_v, sem_i, sem_g, sem_o):
        wid = lax.axis_index("s") * NC + lax.axis_index("c")
        base = wid * b_per_w
        n = n_chunks

        def idx_copy(c, b):
            return pltpu.make_async_copy(
                tok_hbm.at[pl.ds(base + c * CH, CH)], idx_v.at[b], sem_i.at[b]
            )

        def out_copy(c, b):
            return pltpu.make_async_copy(
                rows_v.at[b], out_hbm.at[pl.ds(base + c * CH, CH)], sem_o.at[b]
            )

        def gather(j, b):
            return pltpu.make_async_copy(
                table_hbm.at[idx_v.at[b].at[pl.ds(j * GW, GW)]],
                rows_v.at[b].at[pl.ds(j * GW, GW)],
                sem_g.at[b],
            )

        idx_copy(0, 0).start()
        idx_copy(1, 1).start()
        idx_copy(0, 0).wait()
        for j in range(K):
            gather(j, 0).start()

        def body(it, carry):
            for b in range(NBUF):
                c = 2 * it + b

                @pl.when(c + 1 < n)
                def _():
                    idx_copy(c + 1, 1 - b).wait()

                    @pl.when(c >= 1)
                    def _():
                        out_copy(c - 1, 1 - b).wait()

                    for j in range(K):
                        gather(j, 1 - b).start()

                for j in range(K):
                    gather(j, b).wait()
                out_copy(c, b).start()

                @pl.when(c + 2 < n)
                def _():
                    idx_copy(c + 2, b).start()

            return carry

        lax.fori_loop(0, n // 2, body, 0)

        out_copy(n - 2, n % 2).wait()
        out_copy(n - 1, 1 - n % 2).wait()

    return k(tokens_flat, table)


def kernel(tokens, table):
    B0, S = tokens.shape
    V, D = table.shape
    B = B0 * S
    b_per_w = B // NW
    n_chunks = b_per_w // CH
    assert B % NW == 0 and b_per_w % (CH * NBUF) == 0

    out = _embedding_gather(tokens.reshape(B), table, b_per_w, n_chunks)
    return out.reshape(B0, S, D)
